# baseline (speedup 1.0000x reference)
import jax
import jax.numpy as jnp
from jax import lax
from jax.experimental import pallas as pl
from jax.experimental.pallas import tpu as pltpu
from jax.experimental.pallas import tpu_sc as plsc

_N_NODES = 100000
_BATCH = 1024
_NW = 32
_NPT = _BATCH // _NW
_S1 = _NPT * 10
_S2 = _S1 * 20
_CH = 128
_NCH = _S2 // _CH


def _proj_body(f_ref, w_ref, o_ref):
    o_ref[...] = jnp.dot(f_ref[...], w_ref[...], preferred_element_type=jnp.float32)


def _project(features, w_input):
    return pl.pallas_call(
        _proj_body,
        grid=(10,),
        in_specs=[pl.BlockSpec((10000, 128), lambda i: (i, 0)),
                  pl.BlockSpec((128, 64), lambda i: (0, 0))],
        out_specs=pl.BlockSpec((10000, 64), lambda i: (i, 0)),
        out_shape=jax.ShapeDtypeStruct((_N_NODES, 64), jnp.float32),
    )(features, w_input)


def _sc_body(p_hbm, no_hbm, ni_hbm, n1_hbm, n2_hbm, a0_hbm, a1_hbm,
             nodes_v, nb1_v, s1_v, p0_v, p1_v, nb2_v, s2_v, p2_v,
             agg0_v, agg1_v, zero_v, idx_v, acc_sh,
             sem_1, sem_p, sem_n2, sem_b0, sem_b1):
    cid = lax.axis_index("c")
    sid = lax.axis_index("s")
    wid = sid * 2 + cid
    lane = lax.iota(jnp.int32, 16)
    accbase = pl.multiple_of(sid * _S1, _S1)

    def zbody(r, _):
        for v in range(4):
            zero_v[r, pl.ds(16 * v, 16)] = jnp.zeros((16,), jnp.float32)
        return 0

    lax.fori_loop(0, _S1, zbody, 0)

    for si, nodes_hbm in enumerate((n1_hbm, n2_hbm)):
        base = pl.multiple_of(wid * _NPT, _NPT)
        pltpu.sync_copy(nodes_hbm.at[pl.ds(base, _NPT)], nodes_v)
        c_no1 = pltpu.async_copy(no_hbm.at[nodes_v], nb1_v.at[pl.ds(0, _NPT)], sem_1)
        c_ni1 = pltpu.async_copy(ni_hbm.at[nodes_v], nb1_v.at[pl.ds(_NPT, _NPT)], sem_1)
        pltpu.sync_copy(zero_v, acc_sh.at[pl.ds(accbase, _S1)])
        c_no1.wait()
        c_ni1.wait()

        for t in range(_S1 // 16):
            s = lane + 16 * t
            n = (s.astype(jnp.float32) * (1.0 / 10.0)).astype(jnp.int32)
            j = s - 10 * n
            m = j >= 5
            row = n + jnp.where(m, _NPT, 0)
            col = j - jnp.where(m, 5, 0)
            s1_v[pl.ds(16 * t, 16)] = plsc.load_gather(nb1_v, [row, col])

        c_p0 = pltpu.async_copy(p_hbm.at[nodes_v], p0_v, sem_p)
        c_p1 = [pltpu.async_copy(p_hbm.at[s1_v.at[pl.ds(80 * c, 80)]],
                                 p1_v.at[pl.ds(80 * c, 80)], sem_p)
                for c in range(4)]
        c_n2 = [pltpu.async_copy(no_hbm.at[s1_v.at[pl.ds(80 * c, 80)]],
                                 nb2_v.at[pl.ds(80 * c, 80)], sem_n2)
                for c in range(4)]
        c_i2 = [pltpu.async_copy(ni_hbm.at[s1_v.at[pl.ds(80 * c, 80)]],
                                 nb2_v.at[pl.ds(_S1 + 80 * c, 80)], sem_n2)
                for c in range(4)]
        for h in c_n2 + c_i2:
            h.wait()

        def s2_body(t, _):
            u = lane + 16 * t
            i = (u.astype(jnp.float32) * (1.0 / 20.0)).astype(jnp.int32)
            k = u - 20 * i
            m = k >= 10
            row = i + jnp.where(m, _S1, 0)
            col = k - jnp.where(m, 10, 0)
            s2_v[pl.ds(pl.multiple_of(16 * t, 16), 16)] = (
                plsc.load_gather(nb2_v, [row, col]))
            return 0

        lax.fori_loop(0, _S2 // 16, s2_body, 0)

        def start_chunk(c, b):
            off = pl.multiple_of(c * _CH, _CH)
            return pltpu.async_copy(p_hbm.at[s2_v.at[pl.ds(off, _CH)]],
                                    p2_v.at[b], sem_b0 if b == 0 else sem_b1)

        start_chunk(0, 0)
        start_chunk(1, 1)

        c_p0.wait()
        for h in c_p1:
            h.wait()

        def agg0_body(n, _):
            for v in range(4):
                acc = jnp.zeros((16,), jnp.float32)
                for k in range(10):
                    acc = acc + p1_v[10 * n + k, pl.ds(16 * v, 16)]
                agg0_v[n, pl.ds(16 * v, 16)] = (
                    p0_v[n, pl.ds(16 * v, 16)] + 0.2 * acc)
            return 0

        lax.fori_loop(0, _NPT, agg0_body, 0)
        pltpu.sync_copy(
            agg0_v, a0_hbm.at[pl.ds(pl.multiple_of(si * _BATCH + base, _NPT), _NPT)])

        def pair_body(pr, _):
            for b in range(2):
                c = 2 * pr + b
                sem = sem_b0 if b == 0 else sem_b1
                pltpu.make_async_copy(p_hbm.at[s2_v.at[pl.ds(0, _CH)]],
                                      p2_v.at[b], sem).wait()
                for q in range(_CH // 16):
                    u = lane + (16 * q) + _CH * c
                    g = (u.astype(jnp.float32) * (1.0 / 20.0)).astype(jnp.int32)
                    idx_v[pl.ds(16 * q, 16)] = g + accbase
                pltpu.sync_copy(p2_v.at[b], acc_sh.at[idx_v], add=True)

                @pl.when(c + 2 < _NCH)
                def _():
                    start_chunk(c + 2, b)

            return 0

        lax.fori_loop(0, _NCH // 2, pair_body, 0)

        pltpu.sync_copy(acc_sh.at[pl.ds(accbase, _S1)], agg1_v)

        def fin_body(i, _):
            for v in range(4):
                agg1_v[i, pl.ds(16 * v, 16)] = (
                    p1_v[i, pl.ds(16 * v, 16)]
                    + 0.1 * agg1_v[i, pl.ds(16 * v, 16)])
            return 0

        lax.fori_loop(0, _S1, fin_body, 0)
        pltpu.sync_copy(
            agg1_v,
            a1_hbm.at[pl.ds(pl.multiple_of(si * _BATCH * 10 + wid * _S1, _S1), _S1)])


def _sc_agg(p, neighbors_out, neighbors_in, nodes1, nodes2):
    neighbors_out = jax.lax.slice(neighbors_out, (0, 0), (_N_NODES, 16))
    neighbors_in = jax.lax.slice(neighbors_in, (0, 0), (_N_NODES, 16))
    mesh = plsc.VectorSubcoreMesh(core_axis_name="c", subcore_axis_name="s")
    f = pl.kernel(
        _sc_body,
        out_type=[jax.ShapeDtypeStruct((2 * _BATCH, 64), jnp.float32),
                  jax.ShapeDtypeStruct((2 * _BATCH * 10, 64), jnp.float32)],
        mesh=mesh,
        scratch_types=[
            pltpu.VMEM((_NPT,), jnp.int32),
            pltpu.VMEM((2 * _NPT, 16), jnp.int32),
            pltpu.VMEM((_S1,), jnp.int32),
            pltpu.VMEM((_NPT, 64), jnp.float32),
            pltpu.VMEM((_S1, 64), jnp.float32),
            pltpu.VMEM((2 * _S1, 16), jnp.int32),
            pltpu.VMEM((_S2,), jnp.int32),
            pltpu.VMEM((2, _CH, 64), jnp.float32),
            pltpu.VMEM((_NPT, 64), jnp.float32),
            pltpu.VMEM((_S1, 64), jnp.float32),
            pltpu.VMEM((_S1, 64), jnp.float32),
            pltpu.VMEM((_CH,), jnp.int32),
            pltpu.VMEM_SHARED((16 * _S1, 64), jnp.float32),
            pltpu.SemaphoreType.DMA,
            pltpu.SemaphoreType.DMA,
            pltpu.SemaphoreType.DMA,
            pltpu.SemaphoreType.DMA,
            pltpu.SemaphoreType.DMA,
        ],
        compiler_params=pltpu.CompilerParams(use_tc_tiling_on_sc=False,
                                             needs_layout_passes=False),
    )
    return f(p, neighbors_out, neighbors_in, nodes1, nodes2)


def _epi_body(a0_ref, a1_ref, wm_ref, ws_ref, o_ref):
    h0 = jax.nn.sigmoid(a0_ref[...])
    h1 = jax.nn.sigmoid(a1_ref[...])
    grp = jnp.sum(h1.reshape(2 * _BATCH, 10, 64), axis=1)
    bv = h0 + 0.2 * grp
    m = jax.nn.sigmoid(jnp.dot(bv, wm_ref[...], preferred_element_type=jnp.float32))
    s = jax.nn.sigmoid(jnp.dot(bv, ws_ref[...], preferred_element_type=jnp.float32))
    o_ref[...] = jnp.concatenate(
        [m[:_BATCH], s[:_BATCH], m[_BATCH:], s[_BATCH:]], axis=1)


def _epilogue(a0, a1, w_mean, w_std):
    return pl.pallas_call(
        _epi_body,
        out_shape=jax.ShapeDtypeStruct((_BATCH, 128), jnp.float32),
    )(a0, a1, w_mean, w_std)


def kernel(features, neighbors_out, neighbors_in, nodes1, nodes2, W_input, W_mean, W_std):
    p = _project(features, W_input)
    a0, a1 = _sc_agg(p, neighbors_out, neighbors_in, nodes1, nodes2)
    return _epilogue(a0, a1, W_mean, W_std)


# speedup vs baseline: 1.1209x; 1.1209x over previous
import jax
import jax.numpy as jnp
from jax import lax
from jax.experimental import pallas as pl
from jax.experimental.pallas import tpu as pltpu
from jax.experimental.pallas import tpu_sc as plsc

_N_NODES = 100000
_BATCH = 1024
_NW = 32
_NPT = _BATCH // _NW
_S1 = _NPT * 10
_S2 = _S1 * 20
_CH = 128
_NCH = _S2 // _CH


def _proj_body(f_ref, w_ref, o_ref):
    y = jnp.dot(f_ref[...], w_ref[...], preferred_element_type=jnp.float32)
    y3 = y.reshape(5000, 2, 64)
    o_ref[...] = jnp.concatenate([y3[:, 0, :], y3[:, 1, :]], axis=1)


def _project(features, w_input):
    out = pl.pallas_call(
        _proj_body,
        grid=(10,),
        in_specs=[pl.BlockSpec((10000, 128), lambda i: (i, 0)),
                  pl.BlockSpec((128, 64), lambda i: (0, 0))],
        out_specs=pl.BlockSpec((5000, 128), lambda i: (i, 0)),
        out_shape=jax.ShapeDtypeStruct((_N_NODES // 2, 128), jnp.float32),
    )(features, w_input)
    return jnp.reshape(out, (_N_NODES, 64))


def _sc_body(p_hbm, no_hbm, ni_hbm, n1_hbm, n2_hbm, a0_hbm, a1_hbm,
             nodes_v, nb1_v, s1_v, p0_v, p1_v, nb2_v, s2_v, p2_v,
             agg0_v, agg1_v, zero_v, idx_v, acc_sh,
             sem_1, sem_p, sem_n2, sem_b0, sem_b1):
    cid = lax.axis_index("c")
    sid = lax.axis_index("s")
    wid = sid * 2 + cid
    lane = lax.iota(jnp.int32, 16)
    accbase = pl.multiple_of(sid * _S1, _S1)

    def zbody(r, _):
        for v in range(4):
            zero_v[r, pl.ds(16 * v, 16)] = jnp.zeros((16,), jnp.float32)
        return 0

    lax.fori_loop(0, _S1, zbody, 0)

    for si, nodes_hbm in enumerate((n1_hbm, n2_hbm)):
        base = pl.multiple_of(wid * _NPT, _NPT)
        pltpu.sync_copy(nodes_hbm.at[pl.ds(base, _NPT)], nodes_v)
        c_no1 = pltpu.async_copy(no_hbm.at[nodes_v], nb1_v.at[pl.ds(0, _NPT)], sem_1)
        c_ni1 = pltpu.async_copy(ni_hbm.at[nodes_v], nb1_v.at[pl.ds(_NPT, _NPT)], sem_1)
        pltpu.sync_copy(zero_v, acc_sh.at[pl.ds(accbase, _S1)])
        c_no1.wait()
        c_ni1.wait()

        for t in range(_S1 // 16):
            s = lane + 16 * t
            n = (s.astype(jnp.float32) * (1.0 / 10.0)).astype(jnp.int32)
            j = s - 10 * n
            m = j >= 5
            row = n + jnp.where(m, _NPT, 0)
            col = j - jnp.where(m, 5, 0)
            s1_v[pl.ds(16 * t, 16)] = plsc.load_gather(nb1_v, [row, col])

        c_p0 = pltpu.async_copy(p_hbm.at[nodes_v], p0_v, sem_p)
        c_p1 = [pltpu.async_copy(p_hbm.at[s1_v.at[pl.ds(80 * c, 80)]],
                                 p1_v.at[pl.ds(80 * c, 80)], sem_p)
                for c in range(4)]
        c_n2 = [pltpu.async_copy(no_hbm.at[s1_v.at[pl.ds(80 * c, 80)]],
                                 nb2_v.at[pl.ds(80 * c, 80)], sem_n2)
                for c in range(4)]
        c_i2 = [pltpu.async_copy(ni_hbm.at[s1_v.at[pl.ds(80 * c, 80)]],
                                 nb2_v.at[pl.ds(_S1 + 80 * c, 80)], sem_n2)
                for c in range(4)]
        for h in c_n2 + c_i2:
            h.wait()

        def s2_body(t, _):
            u = lane + 16 * t
            i = (u.astype(jnp.float32) * (1.0 / 20.0)).astype(jnp.int32)
            k = u - 20 * i
            m = k >= 10
            row = i + jnp.where(m, _S1, 0)
            col = k - jnp.where(m, 10, 0)
            s2_v[pl.ds(pl.multiple_of(16 * t, 16), 16)] = (
                plsc.load_gather(nb2_v, [row, col]))
            return 0

        lax.fori_loop(0, _S2 // 16, s2_body, 0)

        def start_chunk(c, b):
            off = pl.multiple_of(c * _CH, _CH)
            return pltpu.async_copy(p_hbm.at[s2_v.at[pl.ds(off, _CH)]],
                                    p2_v.at[b], sem_b0 if b == 0 else sem_b1)

        start_chunk(0, 0)
        start_chunk(1, 1)

        c_p0.wait()
        for h in c_p1:
            h.wait()

        def agg0_body(n, _):
            for v in range(4):
                acc = jnp.zeros((16,), jnp.float32)
                for k in range(10):
                    acc = acc + p1_v[10 * n + k, pl.ds(16 * v, 16)]
                agg0_v[n, pl.ds(16 * v, 16)] = (
                    p0_v[n, pl.ds(16 * v, 16)] + 0.2 * acc)
            return 0

        lax.fori_loop(0, _NPT, agg0_body, 0)
        pltpu.sync_copy(
            agg0_v, a0_hbm.at[pl.ds(pl.multiple_of(si * _BATCH + base, _NPT), _NPT)])

        def pair_body(pr, _):
            for b in range(2):
                c = 2 * pr + b
                sem = sem_b0 if b == 0 else sem_b1
                pltpu.make_async_copy(p_hbm.at[s2_v.at[pl.ds(0, _CH)]],
                                      p2_v.at[b], sem).wait()
                for q in range(_CH // 16):
                    u = lane + (16 * q) + _CH * c
                    g = (u.astype(jnp.float32) * (1.0 / 20.0)).astype(jnp.int32)
                    idx_v[pl.ds(16 * q, 16)] = g + accbase
                pltpu.sync_copy(p2_v.at[b], acc_sh.at[idx_v], add=True)

                @pl.when(c + 2 < _NCH)
                def _():
                    start_chunk(c + 2, b)

            return 0

        lax.fori_loop(0, _NCH // 2, pair_body, 0)

        pltpu.sync_copy(acc_sh.at[pl.ds(accbase, _S1)], agg1_v)

        def fin_body(i, _):
            for v in range(4):
                agg1_v[i, pl.ds(16 * v, 16)] = (
                    p1_v[i, pl.ds(16 * v, 16)]
                    + 0.1 * agg1_v[i, pl.ds(16 * v, 16)])
            return 0

        lax.fori_loop(0, _S1, fin_body, 0)
        pltpu.sync_copy(
            agg1_v,
            a1_hbm.at[pl.ds(pl.multiple_of(si * _BATCH * 10 + wid * _S1, _S1), _S1)])


def _sc_agg(p, neighbors_out, neighbors_in, nodes1, nodes2):
    neighbors_out = jax.lax.slice(neighbors_out, (0, 0), (_N_NODES, 16))
    neighbors_in = jax.lax.slice(neighbors_in, (0, 0), (_N_NODES, 16))
    mesh = plsc.VectorSubcoreMesh(core_axis_name="c", subcore_axis_name="s")
    f = pl.kernel(
        _sc_body,
        out_type=[jax.ShapeDtypeStruct((2 * _BATCH, 64), jnp.float32),
                  jax.ShapeDtypeStruct((2 * _BATCH * 10, 64), jnp.float32)],
        mesh=mesh,
        scratch_types=[
            pltpu.VMEM((_NPT,), jnp.int32),
            pltpu.VMEM((2 * _NPT, 16), jnp.int32),
            pltpu.VMEM((_S1,), jnp.int32),
            pltpu.VMEM((_NPT, 64), jnp.float32),
            pltpu.VMEM((_S1, 64), jnp.float32),
            pltpu.VMEM((2 * _S1, 16), jnp.int32),
            pltpu.VMEM((_S2,), jnp.int32),
            pltpu.VMEM((2, _CH, 64), jnp.float32),
            pltpu.VMEM((_NPT, 64), jnp.float32),
            pltpu.VMEM((_S1, 64), jnp.float32),
            pltpu.VMEM((_S1, 64), jnp.float32),
            pltpu.VMEM((_CH,), jnp.int32),
            pltpu.VMEM_SHARED((16 * _S1, 64), jnp.float32),
            pltpu.SemaphoreType.DMA,
            pltpu.SemaphoreType.DMA,
            pltpu.SemaphoreType.DMA,
            pltpu.SemaphoreType.DMA,
            pltpu.SemaphoreType.DMA,
        ],
        compiler_params=pltpu.CompilerParams(use_tc_tiling_on_sc=False,
                                             needs_layout_passes=False),
    )
    return f(p, neighbors_out, neighbors_in, nodes1, nodes2)


def _epi_body(a0_ref, a1_ref, wm_ref, ws_ref, o_ref):
    h0 = jax.nn.sigmoid(a0_ref[...])
    h1 = jax.nn.sigmoid(a1_ref[...])
    grp = jnp.sum(h1.reshape(2 * _BATCH, 10, 64), axis=1)
    bv = h0 + 0.2 * grp
    m = jax.nn.sigmoid(jnp.dot(bv, wm_ref[...], preferred_element_type=jnp.float32))
    s = jax.nn.sigmoid(jnp.dot(bv, ws_ref[...], preferred_element_type=jnp.float32))
    o_ref[...] = jnp.concatenate(
        [m[:_BATCH], s[:_BATCH], m[_BATCH:], s[_BATCH:]], axis=1)


def _epilogue(a0, a1, w_mean, w_std):
    return pl.pallas_call(
        _epi_body,
        out_shape=jax.ShapeDtypeStruct((_BATCH, 128), jnp.float32),
    )(a0, a1, w_mean, w_std)


def kernel(features, neighbors_out, neighbors_in, nodes1, nodes2, W_input, W_mean, W_std):
    p = _project(features, W_input)
    a0, a1 = _sc_agg(p, neighbors_out, neighbors_in, nodes1, nodes2)
    return _epilogue(a0, a1, W_mean, W_std)


# speedup vs baseline: 1.1497x; 1.0257x over previous
import jax
import jax.numpy as jnp
from jax import lax
from jax.experimental import pallas as pl
from jax.experimental.pallas import tpu as pltpu
from jax.experimental.pallas import tpu_sc as plsc

_N_NODES = 100000
_BATCH = 1024
_NW = 32
_NPT = _BATCH // _NW
_S1 = _NPT * 10
_S2 = _S1 * 20
_CH = 80
_NCH = _S2 // _CH
_NB = 4


def _proj_body(f_ref, w_ref, o_ref):
    y = jnp.dot(f_ref[...], w_ref[...], preferred_element_type=jnp.float32)
    y3 = y.reshape(5000, 2, 64)
    o_ref[...] = jnp.concatenate([y3[:, 0, :], y3[:, 1, :]], axis=1)


def _project(features, w_input):
    out = pl.pallas_call(
        _proj_body,
        grid=(10,),
        in_specs=[pl.BlockSpec((10000, 128), lambda i: (i, 0)),
                  pl.BlockSpec((128, 64), lambda i: (0, 0))],
        out_specs=pl.BlockSpec((5000, 128), lambda i: (i, 0)),
        out_shape=jax.ShapeDtypeStruct((_N_NODES // 2, 128), jnp.float32),
    )(features, w_input)
    return jnp.reshape(out, (_N_NODES, 64))


def _sc_body(p_hbm, no_hbm, ni_hbm, n1_hbm, n2_hbm, a0_hbm, a1_hbm,
             nodes_v, nb1_v, s1_v, p0_v, p1_v, nb2_v, s2_v, p2_v,
             agg0_v, agg1_v, zero_v, idx_v, acc_sh,
             sem_1, sem_p, sem_n2, sem_g, sem_s):
    cid = lax.axis_index("c")
    sid = lax.axis_index("s")
    wid = sid * 2 + cid
    lane = lax.iota(jnp.int32, 16)
    accbase = pl.multiple_of(sid * _S1, _S1)

    def zbody(r, _):
        for v in range(4):
            zero_v[r, pl.ds(16 * v, 16)] = jnp.zeros((16,), jnp.float32)
        return 0

    lax.fori_loop(0, _S1, zbody, 0)

    for si, nodes_hbm in enumerate((n1_hbm, n2_hbm)):
        base = pl.multiple_of(wid * _NPT, _NPT)
        pltpu.sync_copy(nodes_hbm.at[pl.ds(base, _NPT)], nodes_v)
        c_no1 = pltpu.async_copy(no_hbm.at[nodes_v], nb1_v.at[pl.ds(0, _NPT)], sem_1)
        c_ni1 = pltpu.async_copy(ni_hbm.at[nodes_v], nb1_v.at[pl.ds(_NPT, _NPT)], sem_1)
        pltpu.sync_copy(zero_v, acc_sh.at[pl.ds(accbase, _S1)])
        c_no1.wait()
        c_ni1.wait()

        for t in range(_S1 // 16):
            s = lane + 16 * t
            n = (s.astype(jnp.float32) * (1.0 / 10.0)).astype(jnp.int32)
            j = s - 10 * n
            m = j >= 5
            row = n + jnp.where(m, _NPT, 0)
            col = j - jnp.where(m, 5, 0)
            s1_v[pl.ds(16 * t, 16)] = plsc.load_gather(nb1_v, [row, col])

        c_p0 = pltpu.async_copy(p_hbm.at[nodes_v], p0_v, sem_p)
        c_p1 = [pltpu.async_copy(p_hbm.at[s1_v.at[pl.ds(80 * c, 80)]],
                                 p1_v.at[pl.ds(80 * c, 80)], sem_p)
                for c in range(4)]
        c_n2 = [pltpu.async_copy(no_hbm.at[s1_v.at[pl.ds(80 * c, 80)]],
                                 nb2_v.at[pl.ds(80 * c, 80)], sem_n2)
                for c in range(4)]
        c_i2 = [pltpu.async_copy(ni_hbm.at[s1_v.at[pl.ds(80 * c, 80)]],
                                 nb2_v.at[pl.ds(_S1 + 80 * c, 80)], sem_n2)
                for c in range(4)]
        for h in c_n2 + c_i2:
            h.wait()

        def s2_body(t, _):
            u = lane + 16 * t
            i = (u.astype(jnp.float32) * (1.0 / 20.0)).astype(jnp.int32)
            k = u - 20 * i
            m = k >= 10
            row = i + jnp.where(m, _S1, 0)
            col = k - jnp.where(m, 10, 0)
            s2_v[pl.ds(pl.multiple_of(16 * t, 16), 16)] = (
                plsc.load_gather(nb2_v, [row, col]))
            return 0

        lax.fori_loop(0, _S2 // 16, s2_body, 0)

        def start_chunk(c, b):
            off = pl.multiple_of(c * _CH, _CH)
            return pltpu.async_copy(p_hbm.at[s2_v.at[pl.ds(off, _CH)]],
                                    p2_v.at[b], sem_g.at[b])

        for b in range(_NB - 1):
            start_chunk(b, b)

        c_p0.wait()
        for h in c_p1:
            h.wait()

        def agg0_body(n, _):
            for v in range(4):
                acc = jnp.zeros((16,), jnp.float32)
                for k in range(10):
                    acc = acc + p1_v[10 * n + k, pl.ds(16 * v, 16)]
                agg0_v[n, pl.ds(16 * v, 16)] = (
                    p0_v[n, pl.ds(16 * v, 16)] + 0.2 * acc)
            return 0

        lax.fori_loop(0, _NPT, agg0_body, 0)
        pltpu.sync_copy(
            agg0_v, a0_hbm.at[pl.ds(pl.multiple_of(si * _BATCH + base, _NPT), _NPT)])

        def scatter_wait(b):
            pltpu.make_async_copy(p2_v.at[b], acc_sh.at[idx_v.at[b]],
                                  sem_s.at[b]).wait()

        def ring_body(pr, _):
            for b in range(_NB):
                c = _NB * pr + b
                pltpu.make_async_copy(p_hbm.at[s2_v.at[pl.ds(0, _CH)]],
                                      p2_v.at[b], sem_g.at[b]).wait()
                for q in range(_CH // 16):
                    u = lane + (16 * q) + _CH * c
                    g = (u.astype(jnp.float32) * (1.0 / 20.0)).astype(jnp.int32)
                    idx_v[b, pl.ds(16 * q, 16)] = g + accbase
                pltpu.async_copy(p2_v.at[b], acc_sh.at[idx_v.at[b]],
                                 sem_s.at[b], add=True)
                nb = (b + _NB - 1) % _NB

                @pl.when(c + _NB - 1 < _NCH)
                def _():
                    @pl.when(c >= 1)
                    def _():
                        scatter_wait(nb)

                    start_chunk(c + _NB - 1, nb)

            return 0

        lax.fori_loop(0, _NCH // _NB, ring_body, 0)
        for b in range(_NB):
            scatter_wait(b)

        pltpu.sync_copy(acc_sh.at[pl.ds(accbase, _S1)], agg1_v)

        def fin_body(i, _):
            for v in range(4):
                agg1_v[i, pl.ds(16 * v, 16)] = (
                    p1_v[i, pl.ds(16 * v, 16)]
                    + 0.1 * agg1_v[i, pl.ds(16 * v, 16)])
            return 0

        lax.fori_loop(0, _S1, fin_body, 0)
        pltpu.sync_copy(
            agg1_v,
            a1_hbm.at[pl.ds(pl.multiple_of(si * _BATCH * 10 + wid * _S1, _S1), _S1)])


def _sc_agg(p, neighbors_out, neighbors_in, nodes1, nodes2):
    neighbors_out = jax.lax.slice(neighbors_out, (0, 0), (_N_NODES, 16))
    neighbors_in = jax.lax.slice(neighbors_in, (0, 0), (_N_NODES, 16))
    mesh = plsc.VectorSubcoreMesh(core_axis_name="c", subcore_axis_name="s")
    f = pl.kernel(
        _sc_body,
        out_type=[jax.ShapeDtypeStruct((2 * _BATCH, 64), jnp.float32),
                  jax.ShapeDtypeStruct((2 * _BATCH * 10, 64), jnp.float32)],
        mesh=mesh,
        scratch_types=[
            pltpu.VMEM((_NPT,), jnp.int32),
            pltpu.VMEM((2 * _NPT, 16), jnp.int32),
            pltpu.VMEM((_S1,), jnp.int32),
            pltpu.VMEM((_NPT, 64), jnp.float32),
            pltpu.VMEM((_S1, 64), jnp.float32),
            pltpu.VMEM((2 * _S1, 16), jnp.int32),
            pltpu.VMEM((_S2,), jnp.int32),
            pltpu.VMEM((_NB, _CH, 64), jnp.float32),
            pltpu.VMEM((_NPT, 64), jnp.float32),
            pltpu.VMEM((_S1, 64), jnp.float32),
            pltpu.VMEM((_S1, 64), jnp.float32),
            pltpu.VMEM((_NB, _CH), jnp.int32),
            pltpu.VMEM_SHARED((16 * _S1, 64), jnp.float32),
            pltpu.SemaphoreType.DMA,
            pltpu.SemaphoreType.DMA,
            pltpu.SemaphoreType.DMA,
            pltpu.SemaphoreType.DMA((_NB,)),
            pltpu.SemaphoreType.DMA((_NB,)),
        ],
        compiler_params=pltpu.CompilerParams(use_tc_tiling_on_sc=False,
                                             needs_layout_passes=False),
    )
    return f(p, neighbors_out, neighbors_in, nodes1, nodes2)


def _epi_body(a0_ref, a1_ref, wm_ref, ws_ref, o_ref):
    h0 = jax.nn.sigmoid(a0_ref[...])
    h1 = jax.nn.sigmoid(a1_ref[...])
    grp = jnp.sum(h1.reshape(2 * _BATCH, 10, 64), axis=1)
    bv = h0 + 0.2 * grp
    m = jax.nn.sigmoid(jnp.dot(bv, wm_ref[...], preferred_element_type=jnp.float32))
    s = jax.nn.sigmoid(jnp.dot(bv, ws_ref[...], preferred_element_type=jnp.float32))
    o_ref[...] = jnp.concatenate(
        [m[:_BATCH], s[:_BATCH], m[_BATCH:], s[_BATCH:]], axis=1)


def _epilogue(a0, a1, w_mean, w_std):
    return pl.pallas_call(
        _epi_body,
        out_shape=jax.ShapeDtypeStruct((_BATCH, 128), jnp.float32),
    )(a0, a1, w_mean, w_std)


def kernel(features, neighbors_out, neighbors_in, nodes1, nodes2, W_input, W_mean, W_std):
    p = _project(features, W_input)
    a0, a1 = _sc_agg(p, neighbors_out, neighbors_in, nodes1, nodes2)
    return _epilogue(a0, a1, W_mean, W_std)


# speedup vs baseline: 1.1749x; 1.0219x over previous
import jax
import jax.numpy as jnp
from jax import lax
from jax.experimental import pallas as pl
from jax.experimental.pallas import tpu as pltpu
from jax.experimental.pallas import tpu_sc as plsc

_N_NODES = 100000
_BATCH = 1024
_NW = 32
_NPT = _BATCH // _NW
_S1 = _NPT * 10
_S2 = _S1 * 20
_CH = 80
_NCH = _S2 // _CH
_NB = 5


def _proj_body(f_ref, w_ref, o_ref):
    y = jnp.dot(f_ref[...], w_ref[...], preferred_element_type=jnp.float32)
    y3 = y.reshape(5000, 2, 64)
    o_ref[...] = jnp.concatenate([y3[:, 0, :], y3[:, 1, :]], axis=1)


def _project(features, w_input):
    out = pl.pallas_call(
        _proj_body,
        grid=(10,),
        in_specs=[pl.BlockSpec((10000, 128), lambda i: (i, 0)),
                  pl.BlockSpec((128, 64), lambda i: (0, 0))],
        out_specs=pl.BlockSpec((5000, 128), lambda i: (i, 0)),
        out_shape=jax.ShapeDtypeStruct((_N_NODES // 2, 128), jnp.float32),
    )(features, w_input)
    return jnp.reshape(out, (_N_NODES, 64))


def _sc_body(p_hbm, no_hbm, ni_hbm, n1_hbm, n2_hbm, a0_hbm, a1_hbm,
             nodes_v, nb1_v, s1_v, p0_v, p1_v, nb2_v, s2_v, p2_v,
             agg0_v, agg1_v, zero_v, idx_v, acc_sh,
             sem_1, sem_p, sem_n2, sem_g, sem_s):
    cid = lax.axis_index("c")
    sid = lax.axis_index("s")
    wid = sid * 2 + cid
    lane = lax.iota(jnp.int32, 16)
    accbase = pl.multiple_of(sid * _S1, _S1)

    def zbody(r, _):
        for v in range(4):
            zero_v[r, pl.ds(16 * v, 16)] = jnp.zeros((16,), jnp.float32)
        return 0

    lax.fori_loop(0, _S1, zbody, 0)

    for si, nodes_hbm in enumerate((n1_hbm, n2_hbm)):
        base = pl.multiple_of(wid * _NPT, _NPT)
        pltpu.sync_copy(nodes_hbm.at[pl.ds(base, _NPT)], nodes_v)
        c_no1 = pltpu.async_copy(no_hbm.at[nodes_v], nb1_v.at[pl.ds(0, _NPT)], sem_1)
        c_ni1 = pltpu.async_copy(ni_hbm.at[nodes_v], nb1_v.at[pl.ds(_NPT, _NPT)], sem_1)
        pltpu.sync_copy(zero_v, acc_sh.at[pl.ds(accbase, _S1)])
        c_no1.wait()
        c_ni1.wait()

        for t in range(_S1 // 16):
            s = lane + 16 * t
            n = (s.astype(jnp.float32) * (1.0 / 10.0)).astype(jnp.int32)
            j = s - 10 * n
            m = j >= 5
            row = n + jnp.where(m, _NPT, 0)
            col = j - jnp.where(m, 5, 0)
            s1_v[pl.ds(16 * t, 16)] = plsc.load_gather(nb1_v, [row, col])

        c_p0 = pltpu.async_copy(p_hbm.at[nodes_v], p0_v, sem_p)
        c_p1 = [pltpu.async_copy(p_hbm.at[s1_v.at[pl.ds(80 * c, 80)]],
                                 p1_v.at[pl.ds(80 * c, 80)], sem_p)
                for c in range(4)]
        c_n2 = [pltpu.async_copy(no_hbm.at[s1_v.at[pl.ds(80 * c, 80)]],
                                 nb2_v.at[pl.ds(80 * c, 80)], sem_n2)
                for c in range(4)]
        c_i2 = [pltpu.async_copy(ni_hbm.at[s1_v.at[pl.ds(80 * c, 80)]],
                                 nb2_v.at[pl.ds(_S1 + 80 * c, 80)], sem_n2)
                for c in range(4)]
        for h in c_n2 + c_i2:
            h.wait()

        def s2_body(t, _):
            u = lane + 16 * t
            i = (u.astype(jnp.float32) * (1.0 / 20.0)).astype(jnp.int32)
            k = u - 20 * i
            m = k >= 10
            row = i + jnp.where(m, _S1, 0)
            col = k - jnp.where(m, 10, 0)
            s2_v[pl.ds(pl.multiple_of(16 * t, 16), 16)] = (
                plsc.load_gather(nb2_v, [row, col]))
            return 0

        lax.fori_loop(0, _S2 // 16, s2_body, 0)

        def start_chunk(c, b):
            off = pl.multiple_of(c * _CH, _CH)
            return pltpu.async_copy(p_hbm.at[s2_v.at[pl.ds(off, _CH)]],
                                    p2_v.at[b], sem_g.at[b])

        for b in range(_NB - 1):
            start_chunk(b, b)

        c_p0.wait()
        for h in c_p1:
            h.wait()

        def agg0_body(n, _):
            for v in range(4):
                acc = jnp.zeros((16,), jnp.float32)
                for k in range(10):
                    acc = acc + p1_v[10 * n + k, pl.ds(16 * v, 16)]
                agg0_v[n, pl.ds(16 * v, 16)] = (
                    p0_v[n, pl.ds(16 * v, 16)] + 0.2 * acc)
            return 0

        lax.fori_loop(0, _NPT, agg0_body, 0)
        pltpu.sync_copy(
            agg0_v, a0_hbm.at[pl.ds(pl.multiple_of(si * _BATCH + base, _NPT), _NPT)])

        def scatter_wait(b):
            pltpu.make_async_copy(p2_v.at[b], acc_sh.at[idx_v.at[b]],
                                  sem_s.at[b]).wait()

        def ring_body(pr, _):
            for b in range(_NB):
                c = _NB * pr + b
                pltpu.make_async_copy(p_hbm.at[s2_v.at[pl.ds(0, _CH)]],
                                      p2_v.at[b], sem_g.at[b]).wait()
                for q in range(_CH // 16):
                    u = lane + (16 * q) + _CH * c
                    g = (u.astype(jnp.float32) * (1.0 / 20.0)).astype(jnp.int32)
                    idx_v[b, pl.ds(16 * q, 16)] = g + accbase
                pltpu.async_copy(p2_v.at[b], acc_sh.at[idx_v.at[b]],
                                 sem_s.at[b], add=True)
                nb = (b + _NB - 1) % _NB

                @pl.when(c + _NB - 1 < _NCH)
                def _():
                    @pl.when(c >= 1)
                    def _():
                        scatter_wait(nb)

                    start_chunk(c + _NB - 1, nb)

            return 0

        lax.fori_loop(0, _NCH // _NB, ring_body, 0)
        for b in range(_NB):
            scatter_wait(b)

        pltpu.sync_copy(acc_sh.at[pl.ds(accbase, _S1)], agg1_v)

        def fin_body(i, _):
            for v in range(4):
                agg1_v[i, pl.ds(16 * v, 16)] = (
                    p1_v[i, pl.ds(16 * v, 16)]
                    + 0.1 * agg1_v[i, pl.ds(16 * v, 16)])
            return 0

        lax.fori_loop(0, _S1, fin_body, 0)
        pltpu.sync_copy(
            agg1_v,
            a1_hbm.at[pl.ds(pl.multiple_of(si * _BATCH * 10 + wid * _S1, _S1), _S1)])


def _sc_agg(p, neighbors_out, neighbors_in, nodes1, nodes2):
    neighbors_out = jax.lax.slice(neighbors_out, (0, 0), (_N_NODES, 16))
    neighbors_in = jax.lax.slice(neighbors_in, (0, 0), (_N_NODES, 16))
    mesh = plsc.VectorSubcoreMesh(core_axis_name="c", subcore_axis_name="s")
    f = pl.kernel(
        _sc_body,
        out_type=[jax.ShapeDtypeStruct((2 * _BATCH, 64), jnp.float32),
                  jax.ShapeDtypeStruct((2 * _BATCH * 10, 64), jnp.float32)],
        mesh=mesh,
        scratch_types=[
            pltpu.VMEM((_NPT,), jnp.int32),
            pltpu.VMEM((2 * _NPT, 16), jnp.int32),
            pltpu.VMEM((_S1,), jnp.int32),
            pltpu.VMEM((_NPT, 64), jnp.float32),
            pltpu.VMEM((_S1, 64), jnp.float32),
            pltpu.VMEM((2 * _S1, 16), jnp.int32),
            pltpu.VMEM((_S2,), jnp.int32),
            pltpu.VMEM((_NB, _CH, 64), jnp.float32),
            pltpu.VMEM((_NPT, 64), jnp.float32),
            pltpu.VMEM((_S1, 64), jnp.float32),
            pltpu.VMEM((_S1, 64), jnp.float32),
            pltpu.VMEM((_NB, _CH), jnp.int32),
            pltpu.VMEM_SHARED((16 * _S1, 64), jnp.float32),
            pltpu.SemaphoreType.DMA,
            pltpu.SemaphoreType.DMA,
            pltpu.SemaphoreType.DMA,
            pltpu.SemaphoreType.DMA((_NB,)),
            pltpu.SemaphoreType.DMA((_NB,)),
        ],
        compiler_params=pltpu.CompilerParams(use_tc_tiling_on_sc=False,
                                             needs_layout_passes=False),
    )
    return f(p, neighbors_out, neighbors_in, nodes1, nodes2)


def _epi_body(a0_ref, a1_ref, wm_ref, ws_ref, o_ref):
    h0 = jax.nn.sigmoid(a0_ref[...])
    h1 = jax.nn.sigmoid(a1_ref[...])
    grp = jnp.sum(h1.reshape(2 * _BATCH, 10, 64), axis=1)
    bv = h0 + 0.2 * grp
    m = jax.nn.sigmoid(jnp.dot(bv, wm_ref[...], preferred_element_type=jnp.float32))
    s = jax.nn.sigmoid(jnp.dot(bv, ws_ref[...], preferred_element_type=jnp.float32))
    o_ref[...] = jnp.concatenate(
        [m[:_BATCH], s[:_BATCH], m[_BATCH:], s[_BATCH:]], axis=1)


def _epilogue(a0, a1, w_mean, w_std):
    return pl.pallas_call(
        _epi_body,
        out_shape=jax.ShapeDtypeStruct((_BATCH, 128), jnp.float32),
    )(a0, a1, w_mean, w_std)


def kernel(features, neighbors_out, neighbors_in, nodes1, nodes2, W_input, W_mean, W_std):
    p = _project(features, W_input)
    a0, a1 = _sc_agg(p, neighbors_out, neighbors_in, nodes1, nodes2)
    return _epilogue(a0, a1, W_mean, W_std)


# speedup vs baseline: 1.1881x; 1.0112x over previous
import jax
import jax.numpy as jnp
from jax import lax
from jax.experimental import pallas as pl
from jax.experimental.pallas import tpu as pltpu
from jax.experimental.pallas import tpu_sc as plsc

_N_NODES = 100000
_BATCH = 1024
_NW = 32
_NPT = _BATCH // _NW
_S1 = _NPT * 10
_S2 = _S1 * 20
_CH = 80
_NCH = _S2 // _CH
_NB = 5


def _proj_body(f_ref, w_ref, o_ref):
    y = jnp.dot(f_ref[...], w_ref[...], preferred_element_type=jnp.float32)
    y3 = y.reshape(5000, 2, 64)
    o_ref[...] = jnp.concatenate([y3[:, 0, :], y3[:, 1, :]], axis=1)


def _project(features, w_input):
    out = pl.pallas_call(
        _proj_body,
        grid=(10,),
        in_specs=[pl.BlockSpec((10000, 128), lambda i: (i, 0)),
                  pl.BlockSpec((128, 64), lambda i: (0, 0))],
        out_specs=pl.BlockSpec((5000, 128), lambda i: (i, 0)),
        out_shape=jax.ShapeDtypeStruct((_N_NODES // 2, 128), jnp.float32),
    )(features, w_input)
    return jnp.reshape(out, (_N_NODES, 64))


def _sc_body(p_hbm, no_hbm, ni_hbm, n1_hbm, n2_hbm, a0_hbm, a1_hbm,
             nodes_v, nb1_v, s1_v, p0_v, p1_v, nb2_v, s2_v, p2_v,
             agg0_v, agg1_v, zero_v, idx_v, acc_sh,
             sem_1, sem_p, sem_n2, sem_g, sem_s):
    cid = lax.axis_index("c")
    sid = lax.axis_index("s")
    wid = sid * 2 + cid
    lane = lax.iota(jnp.int32, 16)
    accbase = pl.multiple_of(sid * _S1, _S1)

    def zbody(r, _):
        for v in range(4):
            zero_v[r, pl.ds(16 * v, 16)] = jnp.zeros((16,), jnp.float32)
        return 0

    lax.fori_loop(0, _S1, zbody, 0)

    for si, nodes_hbm in enumerate((n1_hbm, n2_hbm)):
        base = pl.multiple_of(wid * _NPT, _NPT)
        pltpu.sync_copy(nodes_hbm.at[pl.ds(base, _NPT)], nodes_v)
        c_no1 = pltpu.async_copy(no_hbm.at[nodes_v], nb1_v.at[pl.ds(0, _NPT)], sem_1)
        c_ni1 = pltpu.async_copy(ni_hbm.at[nodes_v], nb1_v.at[pl.ds(_NPT, _NPT)], sem_1)
        pltpu.sync_copy(zero_v, acc_sh.at[pl.ds(accbase, _S1)])
        c_no1.wait()
        c_ni1.wait()

        for t in range(_S1 // 16):
            s = lane + 16 * t
            n = (s.astype(jnp.float32) * (1.0 / 10.0)).astype(jnp.int32)
            j = s - 10 * n
            m = j >= 5
            row = n + jnp.where(m, _NPT, 0)
            col = j - jnp.where(m, 5, 0)
            s1_v[pl.ds(16 * t, 16)] = plsc.load_gather(nb1_v, [row, col])

        c_p0 = pltpu.async_copy(p_hbm.at[nodes_v], p0_v, sem_p)
        c_p1 = [pltpu.async_copy(p_hbm.at[s1_v.at[pl.ds(80 * c, 80)]],
                                 p1_v.at[pl.ds(80 * c, 80)], sem_p)
                for c in range(4)]
        c_n2 = [pltpu.async_copy(no_hbm.at[s1_v.at[pl.ds(80 * c, 80)]],
                                 nb2_v.at[pl.ds(80 * c, 80)], sem_n2)
                for c in range(4)]
        c_i2 = [pltpu.async_copy(ni_hbm.at[s1_v.at[pl.ds(80 * c, 80)]],
                                 nb2_v.at[pl.ds(_S1 + 80 * c, 80)], sem_n2)
                for c in range(4)]
        for h in c_n2 + c_i2:
            h.wait()

        def build_s2_chunk(c):
            for q in range(_CH // 16):
                u = lane + 16 * q + _CH * c
                i = (u.astype(jnp.float32) * (1.0 / 20.0)).astype(jnp.int32)
                k = u - 20 * i
                m = k >= 10
                row = i + jnp.where(m, _S1, 0)
                col = k - jnp.where(m, 10, 0)
                s2_v[pl.ds(pl.multiple_of(_CH * c + 16 * q, 16), 16)] = (
                    plsc.load_gather(nb2_v, [row, col]))

        def start_chunk(c, b):
            off = pl.multiple_of(c * _CH, _CH)
            return pltpu.async_copy(p_hbm.at[s2_v.at[pl.ds(off, _CH)]],
                                    p2_v.at[b], sem_g.at[b])

        for b in range(_NB - 1):
            build_s2_chunk(b)
            start_chunk(b, b)

        c_p0.wait()
        for h in c_p1:
            h.wait()

        def agg0_body(n, _):
            for v in range(4):
                acc = jnp.zeros((16,), jnp.float32)
                for k in range(10):
                    acc = acc + p1_v[10 * n + k, pl.ds(16 * v, 16)]
                agg0_v[n, pl.ds(16 * v, 16)] = (
                    p0_v[n, pl.ds(16 * v, 16)] + 0.2 * acc)
            return 0

        lax.fori_loop(0, _NPT, agg0_body, 0)
        pltpu.sync_copy(
            agg0_v, a0_hbm.at[pl.ds(pl.multiple_of(si * _BATCH + base, _NPT), _NPT)])

        def scatter_wait(b):
            pltpu.make_async_copy(p2_v.at[b], acc_sh.at[idx_v.at[b]],
                                  sem_s.at[b]).wait()

        def ring_body(pr, _):
            for b in range(_NB):
                c = _NB * pr + b
                pltpu.make_async_copy(p_hbm.at[s2_v.at[pl.ds(0, _CH)]],
                                      p2_v.at[b], sem_g.at[b]).wait()
                for q in range(_CH // 16):
                    u = lane + (16 * q) + _CH * c
                    g = (u.astype(jnp.float32) * (1.0 / 20.0)).astype(jnp.int32)
                    idx_v[b, pl.ds(16 * q, 16)] = g + accbase
                pltpu.async_copy(p2_v.at[b], acc_sh.at[idx_v.at[b]],
                                 sem_s.at[b], add=True)
                nb = (b + _NB - 1) % _NB

                @pl.when(c + _NB - 1 < _NCH)
                def _():
                    @pl.when(c >= 1)
                    def _():
                        scatter_wait(nb)

                    build_s2_chunk(c + _NB - 1)
                    start_chunk(c + _NB - 1, nb)

            return 0

        lax.fori_loop(0, _NCH // _NB, ring_body, 0)
        for b in range(_NB):
            scatter_wait(b)

        pltpu.sync_copy(acc_sh.at[pl.ds(accbase, _S1)], agg1_v)

        def fin_body(i, _):
            for v in range(4):
                agg1_v[i, pl.ds(16 * v, 16)] = (
                    p1_v[i, pl.ds(16 * v, 16)]
                    + 0.1 * agg1_v[i, pl.ds(16 * v, 16)])
            return 0

        lax.fori_loop(0, _S1, fin_body, 0)
        pltpu.sync_copy(
            agg1_v,
            a1_hbm.at[pl.ds(pl.multiple_of(si * _BATCH * 10 + wid * _S1, _S1), _S1)])


def _sc_agg(p, neighbors_out, neighbors_in, nodes1, nodes2):
    neighbors_out = jax.lax.slice(neighbors_out, (0, 0), (_N_NODES, 16))
    neighbors_in = jax.lax.slice(neighbors_in, (0, 0), (_N_NODES, 16))
    mesh = plsc.VectorSubcoreMesh(core_axis_name="c", subcore_axis_name="s")
    f = pl.kernel(
        _sc_body,
        out_type=[jax.ShapeDtypeStruct((2 * _BATCH, 64), jnp.float32),
                  jax.ShapeDtypeStruct((2 * _BATCH * 10, 64), jnp.float32)],
        mesh=mesh,
        scratch_types=[
            pltpu.VMEM((_NPT,), jnp.int32),
            pltpu.VMEM((2 * _NPT, 16), jnp.int32),
            pltpu.VMEM((_S1,), jnp.int32),
            pltpu.VMEM((_NPT, 64), jnp.float32),
            pltpu.VMEM((_S1, 64), jnp.float32),
            pltpu.VMEM((2 * _S1, 16), jnp.int32),
            pltpu.VMEM((_S2,), jnp.int32),
            pltpu.VMEM((_NB, _CH, 64), jnp.float32),
            pltpu.VMEM((_NPT, 64), jnp.float32),
            pltpu.VMEM((_S1, 64), jnp.float32),
            pltpu.VMEM((_S1, 64), jnp.float32),
            pltpu.VMEM((_NB, _CH), jnp.int32),
            pltpu.VMEM_SHARED((16 * _S1, 64), jnp.float32),
            pltpu.SemaphoreType.DMA,
            pltpu.SemaphoreType.DMA,
            pltpu.SemaphoreType.DMA,
            pltpu.SemaphoreType.DMA((_NB,)),
            pltpu.SemaphoreType.DMA((_NB,)),
        ],
        compiler_params=pltpu.CompilerParams(use_tc_tiling_on_sc=False,
                                             needs_layout_passes=False),
    )
    return f(p, neighbors_out, neighbors_in, nodes1, nodes2)


def _epi_body(a0_ref, a1_ref, wm_ref, ws_ref, o_ref):
    h0 = jax.nn.sigmoid(a0_ref[...])
    h1 = jax.nn.sigmoid(a1_ref[...])
    grp = jnp.sum(h1.reshape(2 * _BATCH, 10, 64), axis=1)
    bv = h0 + 0.2 * grp
    m = jax.nn.sigmoid(jnp.dot(bv, wm_ref[...], preferred_element_type=jnp.float32))
    s = jax.nn.sigmoid(jnp.dot(bv, ws_ref[...], preferred_element_type=jnp.float32))
    o_ref[...] = jnp.concatenate(
        [m[:_BATCH], s[:_BATCH], m[_BATCH:], s[_BATCH:]], axis=1)


def _epilogue(a0, a1, w_mean, w_std):
    return pl.pallas_call(
        _epi_body,
        out_shape=jax.ShapeDtypeStruct((_BATCH, 128), jnp.float32),
    )(a0, a1, w_mean, w_std)


def kernel(features, neighbors_out, neighbors_in, nodes1, nodes2, W_input, W_mean, W_std):
    p = _project(features, W_input)
    a0, a1 = _sc_agg(p, neighbors_out, neighbors_in, nodes1, nodes2)
    return _epilogue(a0, a1, W_mean, W_std)
